# baseline (speedup 1.0000x reference)
import functools

import jax
import jax.numpy as jnp
from jax import lax
from jax.experimental import pallas as pl
from jax.experimental.pallas import tpu as pltpu
from jax.experimental.pallas import tpu_sc as plsc


def _copy_body(x_ref, y_ref):
    y_ref[...] = x_ref[...]


def _tc_copy(x):
    N, D = x.shape
    ROWS = 8192
    return pl.pallas_call(
        _copy_body,
        grid=(N // ROWS,),
        in_specs=[pl.BlockSpec((ROWS, D), lambda i: (i, 0))],
        out_specs=pl.BlockSpec((ROWS, D), lambda i: (i, 0)),
        out_shape=jax.ShapeDtypeStruct((N, D), jnp.float32),
    )(x)


def _sc_copy(x):
    N, D = x.shape
    info = plsc.get_sparse_core_info()
    NC, NS = info.num_cores, info.num_subcores
    NW = NC * NS
    rows_per_w = N // NW
    CH = 128
    nchunks = rows_per_w // CH
    mesh = plsc.VectorSubcoreMesh(core_axis_name="c", subcore_axis_name="s")

    @functools.partial(
        pl.kernel,
        out_type=jax.ShapeDtypeStruct((N, D), jnp.float32),
        mesh=mesh,
        scratch_types=[
            pltpu.VMEM((2, CH, D), jnp.float32),
            pltpu.SemaphoreType.DMA,
            pltpu.SemaphoreType.DMA,
        ],
    )
    def k(src, dst, buf, sem_in, sem_out):
        wid = lax.axis_index("s") * NC + lax.axis_index("c")
        base = wid * rows_per_w
        first = pltpu.async_copy(src.at[pl.ds(base, CH)], buf.at[0], sem_in)
        first.wait()
        for i in range(nchunks):
            cur = i % 2
            nxt = (i + 1) % 2
            if i + 1 < nchunks:
                rd = pltpu.async_copy(
                    src.at[pl.ds(base + (i + 1) * CH, CH)], buf.at[nxt], sem_in)
            wr = pltpu.async_copy(
                buf.at[cur], dst.at[pl.ds(base + i * CH, CH)], sem_out)
            wr.wait()
            if i + 1 < nchunks:
                rd.wait()

    return k(x)


def kernel(batch, logits, data, batch_weight, element_weight, level):
    a = _tc_copy(logits)
    b = _sc_copy(data)
    return (jnp.float32(0.0), a, b)


# speedup vs baseline: 1.0423x; 1.0423x over previous
import functools

import jax
import jax.numpy as jnp
from jax import lax
from jax.experimental import pallas as pl
from jax.experimental.pallas import tpu as pltpu
from jax.experimental.pallas import tpu_sc as plsc


def _rowloss_body(D, ew_ref, logits_ref, data_ref, wl_ref, logits_out_ref):
    lg = logits_ref[...]
    logits_out_ref[...] = lg
    diff = lg - data_ref[...]
    row = jnp.sum(diff * diff, axis=1) * (1.0 / D)
    wl_ref[...] = row * ew_ref[...]


def _tc_rowloss(logits, data, element_weight):
    N, D = logits.shape
    ROWS = 4096
    grid = N // ROWS
    body = functools.partial(_rowloss_body, D)
    return pl.pallas_call(
        body,
        grid=(grid,),
        in_specs=[
            pl.BlockSpec((ROWS,), lambda i: (i,)),
            pl.BlockSpec((ROWS, D), lambda i: (i, 0)),
            pl.BlockSpec((ROWS, D), lambda i: (i, 0)),
        ],
        out_specs=[
            pl.BlockSpec((ROWS,), lambda i: (i,)),
            pl.BlockSpec((ROWS, D), lambda i: (i, 0)),
        ],
        out_shape=[
            jax.ShapeDtypeStruct((N,), jnp.float32),
            jax.ShapeDtypeStruct((N, D), jnp.float32),
        ],
    )(element_weight, logits, data)


def _lane_allsum(v, L):
    lane = lax.iota(jnp.int32, L)
    dnums = lax.GatherDimensionNumbers(
        offset_dims=(), collapsed_slice_dims=(0,), start_index_map=(0,))
    k = 1
    while k < L:
        perm = lane ^ k
        v = v + lax.gather(v, perm[:, None], dnums, slice_sizes=(1,),
                           mode=lax.GatherScatterMode.PROMISE_IN_BOUNDS)
        k *= 2
    return v


def _sc_scatter_mean(wl, batch):
    N = wl.shape[0]
    B = 16
    L = 16
    NS = 16
    per_w = N // NS
    nv = per_w // L
    mesh = plsc.VectorSubcoreMesh(
        core_axis_name="c", subcore_axis_name="s", num_cores=1)

    @functools.partial(
        pl.kernel,
        out_type=jax.ShapeDtypeStruct((NS, 2, B), jnp.float32),
        mesh=mesh,
        scratch_types=[
            pltpu.VMEM((per_w,), jnp.float32),
            pltpu.VMEM((per_w,), jnp.int32),
            pltpu.VMEM((2, B), jnp.float32),
        ],
    )
    def k(wl_hbm, batch_hbm, out_hbm, wl_v, id_v, part):
        sid = lax.axis_index("s")
        base = sid * per_w
        pltpu.sync_copy(wl_hbm.at[pl.ds(base, per_w)], wl_v)
        pltpu.sync_copy(batch_hbm.at[pl.ds(base, per_w)], id_v)
        zeros = jnp.zeros((L,), jnp.float32)

        def step(j, carry):
            accs, accc = carry
            v = wl_v[pl.ds(j * L, L)]
            ids = id_v[pl.ds(j * L, L)]
            new_s = []
            new_c = []
            for b in range(B):
                m = ids == b
                new_s.append(accs[b] + jnp.where(m, v, 0.0))
                new_c.append(accc[b] + jnp.where(m, 1.0, 0.0))
            return (tuple(new_s), tuple(new_c))

        init = (tuple([zeros] * B), tuple([zeros] * B))
        accs, accc = lax.fori_loop(0, nv, step, init)
        lane = lax.iota(jnp.int32, L)
        s_vec = zeros
        c_vec = zeros
        for b in range(B):
            sel = lane == b
            s_vec = jnp.where(sel, _lane_allsum(accs[b], L), s_vec)
            c_vec = jnp.where(sel, _lane_allsum(accc[b], L), c_vec)
        part[0, :] = s_vec
        part[1, :] = c_vec
        pltpu.sync_copy(part, out_hbm.at[sid])

    return k(wl, batch)


def _finalize_body(NS, B, lvl_ref, part_ref, bw_ref, out_ref):
    p = part_ref[...]
    rows = lax.broadcasted_iota(jnp.int32, (2 * NS, B), 0)
    even = (rows % 2) == 0
    s = jnp.sum(jnp.where(even, p, 0.0), axis=0)
    c = jnp.sum(jnp.where(even, 0.0, p), axis=0)
    seg = s / jnp.clip(c, 1.0, None)
    seg = seg * bw_ref[...]
    seg = jnp.clip(seg, 0.0, lvl_ref[0])
    out_ref[0] = jnp.sum(seg) * (1.0 / B)


def _tc_finalize(partials, bw, lvl):
    NS = partials.shape[0]
    B = bw.shape[0]
    flat = partials.reshape(2 * NS, B)
    body = functools.partial(_finalize_body, NS, B)
    return pl.pallas_call(
        body,
        in_specs=[
            pl.BlockSpec(memory_space=pltpu.MemorySpace.SMEM),
            pl.BlockSpec((2 * NS, B), lambda: (0, 0)),
            pl.BlockSpec((B,), lambda: (0,)),
        ],
        out_specs=pl.BlockSpec(memory_space=pltpu.MemorySpace.SMEM),
        out_shape=jax.ShapeDtypeStruct((1,), jnp.float32),
    )(lvl, flat, bw)


def kernel(batch, logits, data, batch_weight, element_weight, level):
    lvl = jnp.asarray(level, jnp.float32).reshape(1)
    wl, logits_out = _tc_rowloss(logits, data, element_weight)
    partials = _sc_scatter_mean(wl, batch)
    loss = _tc_finalize(partials, batch_weight, lvl)
    return (loss[0], logits_out)


# speedup vs baseline: 2.0872x; 2.0025x over previous
import functools

import jax
import jax.numpy as jnp
from jax import lax
from jax.experimental import pallas as pl
from jax.experimental.pallas import tpu as pltpu


def _loss_body(grid, B, D, lvl_ref, batch_ref, ew_ref, bw_ref,
               logits_ref, data_ref, out_ref, logits_out_ref, s_ref, c_ref):
    step = pl.program_id(0)

    @pl.when(step == 0)
    def _init():
        s_ref[...] = jnp.zeros_like(s_ref)
        c_ref[...] = jnp.zeros_like(c_ref)

    lg = logits_ref[...]
    logits_out_ref[...] = lg
    diff = lg - data_ref[...]
    row = jnp.sum(diff * diff, axis=1) * (1.0 / D)
    wl = row * ew_ref[...]
    ids = batch_ref[...]
    rows = ids.shape[0]
    iot = lax.broadcasted_iota(jnp.int32, (B, rows), 0)
    mask = iot == ids[None, :]
    s_ref[0, :] += jnp.sum(jnp.where(mask, wl[None, :], 0.0), axis=1)
    c_ref[0, :] += jnp.sum(mask.astype(jnp.float32), axis=1)

    @pl.when(step == grid - 1)
    def _fin():
        seg = s_ref[0, :] / jnp.clip(c_ref[0, :], 1.0, None)
        seg = seg * bw_ref[...]
        seg = jnp.clip(seg, 0.0, lvl_ref[0])
        out_ref[0] = jnp.sum(seg) * (1.0 / B)


def kernel(batch, logits, data, batch_weight, element_weight, level):
    N, D = logits.shape
    B = batch_weight.shape[0]
    ROWS = 8192
    grid = N // ROWS
    lvl = jnp.asarray(level, jnp.float32).reshape(1)

    body = functools.partial(_loss_body, grid, B, D)

    loss, logits_out = pl.pallas_call(
        body,
        grid=(grid,),
        in_specs=[
            pl.BlockSpec(memory_space=pltpu.MemorySpace.SMEM),
            pl.BlockSpec((ROWS,), lambda i: (i,)),
            pl.BlockSpec((ROWS,), lambda i: (i,)),
            pl.BlockSpec((B,), lambda i: (0,)),
            pl.BlockSpec((ROWS, D), lambda i: (i, 0)),
            pl.BlockSpec((ROWS, D), lambda i: (i, 0)),
        ],
        out_specs=[
            pl.BlockSpec(memory_space=pltpu.MemorySpace.SMEM),
            pl.BlockSpec((ROWS, D), lambda i: (i, 0)),
        ],
        out_shape=[
            jax.ShapeDtypeStruct((1,), jnp.float32),
            jax.ShapeDtypeStruct((N, D), jnp.float32),
        ],
        scratch_shapes=[
            pltpu.VMEM((1, B), jnp.float32),
            pltpu.VMEM((1, B), jnp.float32),
        ],
    )(lvl, batch, element_weight, batch_weight, logits, data)
    return (loss[0], logits_out)
